# baseline (speedup 1.0000x reference)
import jax
import jax.numpy as jnp
from jax.experimental import pallas as pl
from jax.experimental.pallas import tpu as pltpu

N, E, DN, DE = 2048, 8192, 128, 16
DOUT = DN + DE
B = 256


def _leaky(x):
    return jnp.maximum(x, 0.01 * x)


def _copy(hbm, buf, sem, blk, slot):
    return pltpu.make_async_copy(hbm.at[:, pl.ds(blk * B, B)], buf.at[slot],
                                 sem.at[slot])


def _body(nf_ref, ef_ref, adj_hbm, eadj_hbm, wn_ref, we_ref,
          pn_ref, pen_ref, pee_ref, out_ref,
          rn_ref, re_ref, vself_ref, nfx_ref, efx_ref,
          abuf, ebuf, asem, esem):
    i = pl.program_id(0)
    nsteps = pl.num_programs(0)
    slot = jax.lax.rem(i, 2)
    nxt = jax.lax.rem(i + 1, 2)

    @pl.when(i == 0)
    def _prime():
        _copy(adj_hbm, abuf, asem, 0, 0).start()
        _copy(eadj_hbm, ebuf, esem, 0, 0).start()
        nf = nf_ref[...]
        ef = ef_ref[...]
        wn = wn_ref[...]
        we = we_ref[...]
        v_n = jax.lax.dot_general(wn, pn_ref[...], (((1,), (1,)), ((), ())))
        v_bs = jax.lax.dot_general(wn, pen_ref[...], (((1,), (1,)), ((), ())))
        v_be = jax.lax.dot_general(we, pee_ref[...], (((1,), (1,)), ((), ())))
        a_nb = jnp.dot(nf, v_n[:, 1:2])
        b_nb = jnp.dot(ef, v_be)
        rn_ref[...] = jnp.broadcast_to(jnp.exp(-0.99 * a_nb), (N, B)).astype(jnp.bfloat16)
        re_ref[...] = jnp.broadcast_to(jnp.exp(-0.99 * b_nb), (E, B)).astype(jnp.bfloat16)
        vself_ref[...] = jnp.concatenate([v_n[:, 0:1], v_bs], axis=1)
        one_n = jnp.ones((N, 1), jnp.float32)
        one_e = jnp.ones((E, 1), jnp.float32)
        nfx_ref[...] = (jnp.concatenate([jnp.dot(nf, wn), one_n], axis=1)
                        * jnp.exp(a_nb)).astype(jnp.bfloat16)
        efx_ref[...] = (jnp.concatenate([jnp.dot(ef, we), one_e], axis=1)
                        * jnp.exp(b_nb)).astype(jnp.bfloat16)

    @pl.when(i + 1 < nsteps)
    def _prefetch():
        _copy(adj_hbm, abuf, asem, i + 1, nxt).start()
        _copy(eadj_hbm, ebuf, esem, i + 1, nxt).start()

    nf_blk = nf_ref[pl.ds(i * B, B), :]
    selfs = jax.lax.dot_general(vself_ref[...], nf_blk,
                                (((0,), (1,)), ((), ())))
    s_n = jnp.exp(-0.99 * selfs[0:1, :]).astype(jnp.bfloat16)
    s_e = jnp.exp(-0.99 * selfs[1:2, :]).astype(jnp.bfloat16)
    one_bf = jnp.bfloat16(1.0)

    _copy(adj_hbm, abuf, asem, i, slot).wait()
    adj = abuf[slot]
    u_n = (jnp.maximum(rn_ref[...] * s_n, one_bf)
           * adj.astype(jnp.bfloat16))
    cnt_n = jnp.sum(adj, axis=0).astype(jnp.float32)
    num_n = jax.lax.dot_general(u_n, nfx_ref[...], (((0,), (0,)), ((), ())),
                                preferred_element_type=jnp.float32)
    sum_n = num_n[:, DN]
    scl_n = jnp.where(cnt_n > 0.0,
                      1.0 / (sum_n * jnp.maximum(cnt_n, 1.0)), 0.0)
    out_n = _leaky(num_n[:, :DN] * scl_n[:, None])

    _copy(eadj_hbm, ebuf, esem, i, slot).wait()
    eadj = ebuf[slot]
    u_e = (jnp.maximum(re_ref[...] * s_e, one_bf)
           * eadj.astype(jnp.bfloat16))
    cnt_e = jnp.sum(eadj, axis=0).astype(jnp.float32)
    num_e = jax.lax.dot_general(u_e, efx_ref[...], (((0,), (0,)), ((), ())),
                                preferred_element_type=jnp.float32)
    sum_e = num_e[:, DE]
    scl_e = jnp.where(cnt_e > 0.0,
                      1.0 / (sum_e * jnp.maximum(cnt_e, 1.0)), 0.0)
    out_e = _leaky(num_e[:, :DE] * scl_e[:, None])

    out_ref[:, 0:DN] = out_n
    out_ref[:, DN:DOUT] = out_e


@jax.jit
def kernel(node_features, edge_features, adjacency_matrix, edge_adjacency_matrix,
           weight_node, weight_edge, parameter_vector_node, parameter_vector_edge):
    pn = parameter_vector_node.reshape(2, DN)
    pen = parameter_vector_edge[:DN].reshape(1, DN)
    pee = parameter_vector_edge[DN:].reshape(1, DE)
    grid = (N // B,)
    full = lambda shape: pl.BlockSpec(shape, lambda i: (0, 0))
    return pl.pallas_call(
        _body,
        grid=grid,
        in_specs=[
            full((N, DN)),
            full((E, DE)),
            pl.BlockSpec(memory_space=pltpu.MemorySpace.HBM),
            pl.BlockSpec(memory_space=pltpu.MemorySpace.HBM),
            full((DN, DN)),
            full((DE, DE)),
            full((2, DN)),
            full((1, DN)),
            full((1, DE)),
        ],
        out_specs=pl.BlockSpec((B, DOUT), lambda i: (i, 0)),
        out_shape=jax.ShapeDtypeStruct((N, DOUT), jnp.float32),
        scratch_shapes=[
            pltpu.VMEM((N, B), jnp.bfloat16),
            pltpu.VMEM((E, B), jnp.bfloat16),
            pltpu.VMEM((DN, 2), jnp.float32),
            pltpu.VMEM((N, DN + 1), jnp.bfloat16),
            pltpu.VMEM((E, DE + 1), jnp.bfloat16),
            pltpu.VMEM((2, N, B), jnp.int32),
            pltpu.VMEM((2, E, B), jnp.int32),
            pltpu.SemaphoreType.DMA((2,)),
            pltpu.SemaphoreType.DMA((2,)),
        ],
    )(node_features, edge_features, adjacency_matrix, edge_adjacency_matrix,
      weight_node, weight_edge, pn, pen, pee)


# speedup vs baseline: 1.0181x; 1.0181x over previous
import jax
import jax.numpy as jnp
from jax.experimental import pallas as pl
from jax.experimental.pallas import tpu as pltpu

N, E, DN, DE = 2048, 8192, 128, 16
DOUT = DN + DE
B = 256


def _leaky(x):
    return jnp.maximum(x, 0.01 * x)


def _copy(hbm, buf, sem, blk, slot):
    return pltpu.make_async_copy(hbm.at[:, pl.ds(blk * B, B)], buf.at[slot],
                                 sem.at[slot])


def _body(nf_ref, ef_ref, adj_hbm, eadj_hbm, wn_ref, we_ref,
          pn_ref, pen_ref, pee_ref, out_ref,
          rn_ref, re_ref, vself_ref, nfx_ref, efx_ref,
          abuf, ebuf, asem, esem):
    i = pl.program_id(0)
    nsteps = pl.num_programs(0)
    slot = jax.lax.rem(i, 2)
    nxt = jax.lax.rem(i + 1, 2)

    @pl.when(i == 0)
    def _prime():
        _copy(adj_hbm, abuf, asem, 0, 0).start()
        _copy(eadj_hbm, ebuf, esem, 0, 0).start()
        nf = nf_ref[...]
        ef = ef_ref[...]
        wn = wn_ref[...]
        we = we_ref[...]
        v_n = jax.lax.dot_general(wn, pn_ref[...], (((1,), (1,)), ((), ())))
        v_bs = jax.lax.dot_general(wn, pen_ref[...], (((1,), (1,)), ((), ())))
        v_be = jax.lax.dot_general(we, pee_ref[...], (((1,), (1,)), ((), ())))
        a_nb = jnp.dot(nf, v_n[:, 1:2])
        b_nb = jnp.dot(ef, v_be)
        rn_ref[...] = jnp.broadcast_to(jnp.exp(-0.99 * a_nb), (N, B)).astype(jnp.bfloat16)
        re_ref[...] = jnp.broadcast_to(jnp.exp(-0.99 * b_nb), (E, B)).astype(jnp.bfloat16)
        vself_ref[...] = jnp.concatenate([v_n[:, 0:1], v_bs], axis=1)
        one_n = jnp.ones((N, 1), jnp.float32)
        one_e = jnp.ones((E, 1), jnp.float32)
        nfx_ref[...] = (jnp.concatenate([jnp.dot(nf, wn), one_n], axis=1)
                        * jnp.exp(a_nb)).astype(jnp.bfloat16)
        efx_ref[...] = (jnp.concatenate([jnp.dot(ef, we), one_e], axis=1)
                        * jnp.exp(b_nb)).astype(jnp.bfloat16)

    @pl.when(i + 1 < nsteps)
    def _prefetch():
        _copy(adj_hbm, abuf, asem, i + 1, nxt).start()
        _copy(eadj_hbm, ebuf, esem, i + 1, nxt).start()

    _copy(adj_hbm, abuf, asem, i, slot).wait()
    _copy(eadj_hbm, ebuf, esem, i, slot).wait()

    nf_blk = nf_ref[pl.ds(i * B, B), :]
    selfs = jax.lax.dot_general(vself_ref[...], nf_blk,
                                (((0,), (1,)), ((), ())))
    s_n = jnp.exp(-0.99 * selfs[0:1, :]).astype(jnp.bfloat16)
    s_e = jnp.exp(-0.99 * selfs[1:2, :]).astype(jnp.bfloat16)
    one_bf = jnp.bfloat16(1.0)

    adj = abuf[slot]
    u_n = (jnp.maximum(rn_ref[...] * s_n, one_bf)
           * adj.astype(jnp.bfloat16))
    cnt_n = jnp.sum(adj, axis=0).astype(jnp.float32)
    num_n = jax.lax.dot_general(u_n, nfx_ref[...], (((0,), (0,)), ((), ())),
                                preferred_element_type=jnp.float32)
    sum_n = num_n[:, DN]
    scl_n = jnp.where(cnt_n > 0.0,
                      1.0 / (sum_n * jnp.maximum(cnt_n, 1.0)), 0.0)
    out_n = _leaky(num_n[:, :DN] * scl_n[:, None])

    eadj = ebuf[slot]
    u_e = (jnp.maximum(re_ref[...] * s_e, one_bf)
           * eadj.astype(jnp.bfloat16))
    cnt_e = jnp.sum(eadj, axis=0).astype(jnp.float32)
    num_e = jax.lax.dot_general(u_e, efx_ref[...], (((0,), (0,)), ((), ())),
                                preferred_element_type=jnp.float32)
    sum_e = num_e[:, DE]
    scl_e = jnp.where(cnt_e > 0.0,
                      1.0 / (sum_e * jnp.maximum(cnt_e, 1.0)), 0.0)
    out_e = _leaky(num_e[:, :DE] * scl_e[:, None])

    out_ref[:, 0:DN] = out_n
    out_ref[:, DN:DOUT] = out_e


@jax.jit
def kernel(node_features, edge_features, adjacency_matrix, edge_adjacency_matrix,
           weight_node, weight_edge, parameter_vector_node, parameter_vector_edge):
    pn = parameter_vector_node.reshape(2, DN)
    pen = parameter_vector_edge[:DN].reshape(1, DN)
    pee = parameter_vector_edge[DN:].reshape(1, DE)
    grid = (N // B,)
    full = lambda shape: pl.BlockSpec(shape, lambda i: (0, 0))
    return pl.pallas_call(
        _body,
        grid=grid,
        in_specs=[
            full((N, DN)),
            full((E, DE)),
            pl.BlockSpec(memory_space=pltpu.MemorySpace.HBM),
            pl.BlockSpec(memory_space=pltpu.MemorySpace.HBM),
            full((DN, DN)),
            full((DE, DE)),
            full((2, DN)),
            full((1, DN)),
            full((1, DE)),
        ],
        out_specs=pl.BlockSpec((B, DOUT), lambda i: (i, 0)),
        out_shape=jax.ShapeDtypeStruct((N, DOUT), jnp.float32),
        scratch_shapes=[
            pltpu.VMEM((N, B), jnp.bfloat16),
            pltpu.VMEM((E, B), jnp.bfloat16),
            pltpu.VMEM((DN, 2), jnp.float32),
            pltpu.VMEM((N, DN + 1), jnp.bfloat16),
            pltpu.VMEM((E, DE + 1), jnp.bfloat16),
            pltpu.VMEM((2, N, B), jnp.int32),
            pltpu.VMEM((2, E, B), jnp.int32),
            pltpu.SemaphoreType.DMA((2,)),
            pltpu.SemaphoreType.DMA((2,)),
        ],
    )(node_features, edge_features, adjacency_matrix, edge_adjacency_matrix,
      weight_node, weight_edge, pn, pen, pee)
